# baseline (speedup 1.0000x reference)
import functools

import jax
import jax.numpy as jnp
from jax import lax
from jax.experimental import pallas as pl
from jax.experimental.pallas import tpu as pltpu
from jax.experimental.pallas import tpu_sc as plsc

_NC = 2
_NS = 16
_NW = _NC * _NS
_LANES = 16
_CHUNK = 64
_SLOTS = 16


def _make_sc_dots(B, D, K):
    rows_per_w = B // _NW
    n_chunks = rows_per_w // _CHUNK
    n_vr = D // _LANES
    mesh = plsc.VectorSubcoreMesh(core_axis_name="c", subcore_axis_name="s")

    @functools.partial(
        pl.kernel,
        mesh=mesh,
        compiler_params=pltpu.CompilerParams(
            use_tc_tiling_on_sc=False, needs_layout_passes=False
        ),
        out_type=jax.ShapeDtypeStruct((B, _SLOTS, _LANES), jnp.float32),
        scratch_types=(
            [pltpu.VMEM((K, _CHUNK), jnp.int32)]
            + [pltpu.VMEM((_CHUNK, D), jnp.float32) for _ in range(K)]
            + [pltpu.VMEM((_CHUNK, _SLOTS, _LANES), jnp.float32)]
            + [pltpu.SemaphoreType.DMA]
        ),
    )
    def sc(idx_hbm, emb_hbm, w_hbm, out_hbm, idx_v, *rest):
        bufs = rest[:K]
        part_v = rest[K]
        sem = rest[K + 1]
        wid = lax.axis_index("s") * _NC + lax.axis_index("c")
        base_w = wid * rows_per_w

        zeros = jnp.zeros((_LANES,), jnp.float32)

        def zero_body(r, carry):
            for j in range(K - 1, _SLOTS):
                part_v[r, j] = zeros
            return carry

        lax.fori_loop(0, _CHUNK, zero_body, 0)

        def chunk_body(ci, carry):
            base = base_w + ci * _CHUNK
            pltpu.sync_copy(idx_hbm.at[:, pl.ds(base, _CHUNK)], idx_v)
            handles = [pltpu.async_copy(emb_hbm.at[idx_v.at[0]], bufs[0], sem)]
            for j in range(1, K):
                handles.append(
                    pltpu.async_copy(w_hbm.at[idx_v.at[j]], bufs[j], sem)
                )
            for h in handles:
                h.wait()

            def row_body(r, carry2):
                e = [bufs[0][r, pl.ds(k * _LANES, _LANES)] for k in range(n_vr)]
                for j in range(1, K):
                    w0 = bufs[j][r, pl.ds(0, _LANES)]
                    acc = e[0] * w0
                    for k in range(1, n_vr):
                        acc = acc + e[k] * bufs[j][r, pl.ds(k * _LANES, _LANES)]
                    part_v[r, j - 1] = acc
                return carry2

            lax.fori_loop(0, _CHUNK, row_body, 0)
            pltpu.sync_copy(part_v, out_hbm.at[pl.ds(base, _CHUNK)])
            return carry

        lax.fori_loop(0, n_chunks, chunk_body, 0)

    return sc


def _make_tc_loss(B, NEG, BK):
    def body(p_ref, o_ref):
        @pl.when(pl.program_id(0) == 0)
        def _():
            o_ref[...] = jnp.zeros_like(o_ref)

        x = p_ref[...]
        rows = lax.broadcasted_iota(jnp.int32, (_SLOTS * _LANES, _SLOTS), 0)
        cols = lax.broadcasted_iota(jnp.int32, (_SLOTS * _LANES, _SLOTS), 1)
        m = (rows // _LANES == cols).astype(jnp.float32)
        z = jax.lax.dot_general(
            x, m, (((1,), (0,)), ((), ())), preferred_element_type=jnp.float32
        )
        col = lax.broadcasted_iota(jnp.int32, z.shape, 1)
        t = jnp.where(col == 0, z, -z)
        ls = jnp.log(jax.nn.sigmoid(t))
        ls = jnp.where(col <= NEG, ls, 0.0)
        o_ref[...] = o_ref[...] + (-jnp.sum(ls) / jnp.float32(B))

    return pl.pallas_call(
        body,
        grid=(B // BK,),
        in_specs=[pl.BlockSpec((BK, _SLOTS * _LANES), lambda i: (i, 0))],
        out_specs=pl.BlockSpec((1, 1), lambda i: (0, 0)),
        out_shape=jax.ShapeDtypeStruct((1, 1), jnp.float32),
    )


def kernel(x, targets, negatives, emb_table, out_weight):
    B = x.shape[0]
    NEG = negatives.shape[1]
    D = emb_table.shape[1]
    all_idx = jnp.concatenate(
        [
            x.astype(jnp.int32)[None, :],
            targets.astype(jnp.int32)[None, :],
            negatives.astype(jnp.int32).T,
        ],
        axis=0,
    )
    parts = _make_sc_dots(B, D, NEG + 2)(all_idx, emb_table, out_weight)
    parts2d = jnp.reshape(parts, (B, _SLOTS * _LANES))
    loss = _make_tc_loss(B, NEG, 2048)(parts2d)
    return loss[0, 0]
